# baseline (speedup 1.0000x reference)
import functools

import jax
import jax.numpy as jnp
from jax import lax
from jax.experimental import pallas as pl
from jax.experimental.pallas import tpu as pltpu
from jax.experimental.pallas import tpu_sc as plsc

_T = 128
_J = 26
_HJ = 13
_QT = 32


def _tc_index_prep(xr):
    n, c = xr.shape
    pc = 4
    bt = pc * _T
    grid = n // bt

    def body(x_ref, o_ref):
        ci = x_ref[:, 6:32].astype(jnp.int32)
        cit = ci.T
        for cc in range(pc):
            o_ref[cc * _J:(cc + 1) * _J, :] = cit[:, cc * _T:(cc + 1) * _T]

    return pl.pallas_call(
        body,
        grid=(grid,),
        in_specs=[pl.BlockSpec((bt, c), lambda i: (i, 0))],
        out_specs=pl.BlockSpec((pc * _J, 128), lambda i: (i, 0)),
        out_shape=jax.ShapeDtypeStruct((n // _T * _J, 128), jnp.int32),
    )(xr)


def _sc_embedding_bag(idx2, tpad, n_tokens, d):
    info = plsc.get_sparse_core_info()
    nw = info.num_cores * info.num_subcores
    g_total = n_tokens // _T

    @functools.partial(
        pl.kernel,
        out_type=jax.ShapeDtypeStruct((n_tokens, d), jnp.float32),
        mesh=plsc.VectorSubcoreMesh(core_axis_name="c", subcore_axis_name="s"),
        compiler_params=pltpu.CompilerParams(use_tc_tiling_on_sc=False),
        scratch_types=[
            pltpu.VMEM((_J, 128), jnp.int32),
            pltpu.VMEM((_HJ * _QT, 128), jnp.float32),
            pltpu.VMEM((_HJ * _QT, 128), jnp.float32),
            pltpu.VMEM((_T, d), jnp.float32),
            pltpu.SemaphoreType.DMA,
            pltpu.SemaphoreType.DMA,
        ],
    )
    def k(idx_h, table_h, out_h, idx_v, rows_x, rows_y, emb_v, sem_x, sem_y):
        wid = lax.axis_index("c") * info.num_subcores + lax.axis_index("s")
        extra = jnp.where(wid < 16, 1, 0)
        cpw = 12 + extra
        base_chunk = jnp.where(wid < 16, wid * 13, 208 + (wid - 16) * 12)

        bufs = (rows_x, rows_y)
        sems = (sem_x, sem_y)
        units = [(h, p) for h in range(_T // _QT) for p in range(2)]

        def issue(i):
            h, p = units[i]
            buf, sem = bufs[i % 2], sems[i % 2]
            return [
                pltpu.async_copy(
                    table_h.at[idx_v.at[p * _HJ + j, pl.ds(h * _QT, _QT)]],
                    buf.at[pl.ds(j * _QT, _QT)],
                    sem,
                )
                for j in range(_HJ)
            ]

        def accumulate(i):
            h, p = units[i]
            buf = bufs[i % 2]

            def tok(t, carry2):
                for q in range(d // 16):
                    if p == 0:
                        a = buf[t, pl.ds(q * 16, 16)]
                        jj = range(1, _HJ)
                    else:
                        a = emb_v[h * _QT + t, pl.ds(q * 16, 16)]
                        jj = range(_HJ)
                    for j in jj:
                        a = a + buf[j * _QT + t, pl.ds(q * 16, 16)]
                    emb_v[h * _QT + t, pl.ds(q * 16, 16)] = a
                return carry2

            lax.fori_loop(0, _QT, tok, 0)

        def chunk_body(c, carry):
            g = base_chunk + c
            pltpu.sync_copy(idx_h.at[pl.ds(g * _J, _J)], idx_v)
            nu = len(units)
            cps = [None] * nu
            cps[0] = issue(0)
            cps[1] = issue(1)
            for i in range(nu):
                for cp in cps[i]:
                    cp.wait()
                accumulate(i)
                if i + 2 < nu:
                    cps[i + 2] = issue(i + 2)
            pltpu.sync_copy(emb_v, out_h.at[pl.ds(g * _T, _T)])
            return carry

        lax.fori_loop(0, cpw, chunk_body, 0)

    return k(idx2, tpad)


def _tc_dense(xr, emb, row0, nw1, nb1, nw2, nb2, tw1, tb1, tw2, tb2,
              admv, gv, bv):
    n, c = xr.shape
    d = emb.shape[1]
    bt = 1024
    grid = n // bt

    def body(x_ref, e_ref, r0_r, nw1_r, nb1_r, nw2_r, nb2_r, tw1_r, tb1_r,
             tw2_r, tb2_r, adm_r, g_r, b_r, o_ref):
        xs = x_ref[...]
        ci = xs[:, 6:32]
        cnt = jnp.sum(jnp.where(ci == 0.0, 1.0, 0.0), axis=1, keepdims=True)
        emb_b = e_ref[...] - cnt * r0_r[...]

        num = xs[:, 5:6]
        nmask = jnp.isnan(num)
        numc = jnp.where(nmask, 0.0, num)
        h1 = jnp.maximum(numc * nw1_r[...] + nb1_r[...], 0.0)
        no = jnp.dot(h1, nw2_r[...], preferred_element_type=jnp.float32)
        no = jnp.where(nmask, 0.0, no + nb2_r[...])

        td = xs[:, 0:5]
        tmask = jnp.isnan(td[:, 0:1])
        tdc = jnp.where(jnp.isnan(td), 0.0, td)
        acc = tb1_r[...]
        for kk in range(5):
            acc = acc + tdc[:, kk:kk + 1] * tw1_r[kk:kk + 1, :]
        h2 = jnp.maximum(acc, 0.0)
        to = jnp.dot(h2, tw2_r[...], preferred_element_type=jnp.float32)
        to = jnp.where(tmask, 0.0, to + tb2_r[...])

        enc = emb_b + no + to
        nrm = jnp.sqrt(jnp.sum(enc * enc, axis=1, keepdims=True))
        enc = enc / jnp.maximum(nrm, 1e-10)
        enc = enc + xs[:, 32:33] * adm_r[...]
        mu = jnp.mean(enc, axis=1, keepdims=True)
        dev = enc - mu
        var = jnp.mean(dev * dev, axis=1, keepdims=True)
        o_ref[...] = dev * lax.rsqrt(var + 1e-5) * g_r[...] + b_r[...]

    full = lambda shape: pl.BlockSpec(shape, lambda i: (0, 0))
    return pl.pallas_call(
        body,
        grid=(grid,),
        in_specs=[
            pl.BlockSpec((bt, c), lambda i: (i, 0)),
            pl.BlockSpec((bt, d), lambda i: (i, 0)),
            full((1, d)),
            full((1, d)), full((1, d)), full((d, d)), full((1, d)),
            full((5, d)), full((1, d)), full((d, d)), full((1, d)),
            full((1, d)), full((1, d)), full((1, d)),
        ],
        out_specs=pl.BlockSpec((bt, d), lambda i: (i, 0)),
        out_shape=jax.ShapeDtypeStruct((n, d), jnp.float32),
    )(xr, emb, row0, nw1, nb1, nw2, nb2, tw1, tb1, tw2, tb2, admv, gv, bv)


def kernel(x, table, nw1, nb1, nw2, nb2, tw1, tb1, tw2, tb2, adm, gamma, beta):
    b, s, c = x.shape
    n = b * s
    d = table.shape[1]
    xr = x.reshape(n, c)
    tpad = jnp.pad(table, ((0, 0), (0, 128 - d)))
    idx2 = _tc_index_prep(xr)
    emb = _sc_embedding_bag(idx2, tpad, n, d)
    out = _tc_dense(
        xr, emb, table[0:1],
        nw1, nb1.reshape(1, d), nw2, nb2.reshape(1, d),
        tw1, tb1.reshape(1, d), tw2, tb2.reshape(1, d),
        adm.reshape(1, d), gamma.reshape(1, d), beta.reshape(1, d),
    )
    return out.reshape(b, s, d)


# speedup vs baseline: 1.1223x; 1.1223x over previous
import functools

import jax
import jax.numpy as jnp
from jax import lax
from jax.experimental import pallas as pl
from jax.experimental.pallas import tpu as pltpu
from jax.experimental.pallas import tpu_sc as plsc

_T = 128
_J = 26
_HJ = 13


def _tc_index_prep(xr):
    n, c = xr.shape
    pc = 4
    bt = pc * _T
    grid = n // bt

    def body(x_ref, o_ref):
        ci = x_ref[:, 6:32].astype(jnp.int32)
        cit = ci.T
        for cc in range(pc):
            o_ref[cc * _J:(cc + 1) * _J, :] = cit[:, cc * _T:(cc + 1) * _T]

    return pl.pallas_call(
        body,
        grid=(grid,),
        in_specs=[pl.BlockSpec((bt, c), lambda i: (i, 0))],
        out_specs=pl.BlockSpec((pc * _J, 128), lambda i: (i, 0)),
        out_shape=jax.ShapeDtypeStruct((n // _T * _J, 128), jnp.int32),
    )(xr)


def _sc_embedding_bag(idx2, table, n_tokens):
    info = plsc.get_sparse_core_info()
    nw = info.num_cores * info.num_subcores
    d = table.shape[1]
    g_total = n_tokens // _T
    _HT = _T // 2

    @functools.partial(
        pl.kernel,
        out_type=jax.ShapeDtypeStruct((n_tokens, d), jnp.float32),
        mesh=plsc.VectorSubcoreMesh(core_axis_name="c", subcore_axis_name="s"),
        compiler_params=pltpu.CompilerParams(use_tc_tiling_on_sc=False),
        scratch_types=[
            pltpu.VMEM((_J, 128), jnp.int32),
            pltpu.VMEM((_HJ * _HT, d), jnp.float32),
            pltpu.VMEM((_HJ * _HT, d), jnp.float32),
            pltpu.VMEM((_T, d), jnp.float32),
            pltpu.SemaphoreType.DMA,
            pltpu.SemaphoreType.DMA,
        ],
    )
    def k(idx_h, table_h, out_h, idx_v, rows_x, rows_y, emb_v, sem_x, sem_y):
        wid = lax.axis_index("c") * info.num_subcores + lax.axis_index("s")
        extra = jnp.where(wid < 16, 1, 0)
        cpw = 12 + extra
        base_chunk = jnp.where(wid < 16, wid * 13, 208 + (wid - 16) * 12)

        def issue(buf, sem, h, p):
            return [
                pltpu.async_copy(
                    table_h.at[idx_v.at[p * _HJ + j, pl.ds(h * _HT, _HT)]],
                    buf.at[pl.ds(j * _HT, _HT)],
                    sem,
                )
                for j in range(_HJ)
            ]

        def accumulate(buf, h, p):
            def tok(t, carry2):
                for q in range(d // 16):
                    if p == 0:
                        a = buf[t, pl.ds(q * 16, 16)]
                        jj = range(1, _HJ)
                    else:
                        a = emb_v[h * _HT + t, pl.ds(q * 16, 16)]
                        jj = range(_HJ)
                    for j in jj:
                        a = a + buf[j * _HT + t, pl.ds(q * 16, 16)]
                    emb_v[h * _HT + t, pl.ds(q * 16, 16)] = a
                return carry2

            lax.fori_loop(0, _HT, tok, 0)

        def chunk_body(c, carry):
            g = base_chunk + c
            pltpu.sync_copy(idx_h.at[pl.ds(g * _J, _J)], idx_v)
            cps0 = issue(rows_x, sem_x, 0, 0)
            cps1 = issue(rows_y, sem_y, 0, 1)
            for cp in cps0:
                cp.wait()
            accumulate(rows_x, 0, 0)
            cps2 = issue(rows_x, sem_x, 1, 0)
            for cp in cps1:
                cp.wait()
            accumulate(rows_y, 0, 1)
            cps3 = issue(rows_y, sem_y, 1, 1)
            for cp in cps2:
                cp.wait()
            accumulate(rows_x, 1, 0)
            for cp in cps3:
                cp.wait()
            accumulate(rows_y, 1, 1)
            pltpu.sync_copy(emb_v, out_h.at[pl.ds(g * _T, _T)])
            return carry

        lax.fori_loop(0, cpw, chunk_body, 0)

    return k(idx2, table)


def _tc_dense(xr, emb, row0, nw1, nb1, nw2, nb2, tw1, tb1, tw2, tb2,
              admv, gv, bv):
    n, c = xr.shape
    d = emb.shape[1]
    bt = 1024
    grid = n // bt

    def body(x_ref, e_ref, r0_r, nw1_r, nb1_r, nw2_r, nb2_r, tw1_r, tb1_r,
             tw2_r, tb2_r, adm_r, g_r, b_r, o_ref):
        xs = x_ref[...]
        ci = xs[:, 6:32]
        cnt = jnp.sum(jnp.where(ci == 0.0, 1.0, 0.0), axis=1, keepdims=True)
        emb_b = e_ref[...] - cnt * r0_r[...]

        num = xs[:, 5:6]
        nmask = jnp.isnan(num)
        numc = jnp.where(nmask, 0.0, num)
        h1 = jnp.maximum(numc * nw1_r[...] + nb1_r[...], 0.0)
        no = jnp.dot(h1, nw2_r[...], preferred_element_type=jnp.float32)
        no = jnp.where(nmask, 0.0, no + nb2_r[...])

        td8 = xs[:, 0:8]
        tmask = jnp.isnan(td8[:, 0:1])
        tdc8 = jnp.where(jnp.isnan(td8), 0.0, td8)
        acc = jnp.dot(tdc8, tw1_r[...],
                      preferred_element_type=jnp.float32) + tb1_r[...]
        h2 = jnp.maximum(acc, 0.0)
        to = jnp.dot(h2, tw2_r[...], preferred_element_type=jnp.float32)
        to = jnp.where(tmask, 0.0, to + tb2_r[...])

        enc = emb_b + no + to
        nrm = jnp.sqrt(jnp.sum(enc * enc, axis=1, keepdims=True))
        enc = enc / jnp.maximum(nrm, 1e-10)
        enc = enc + xs[:, 32:33] * adm_r[...]
        mu = jnp.mean(enc, axis=1, keepdims=True)
        dev = enc - mu
        var = jnp.mean(dev * dev, axis=1, keepdims=True)
        o_ref[...] = dev * lax.rsqrt(var + 1e-5) * g_r[...] + b_r[...]

    full = lambda shape: pl.BlockSpec(shape, lambda i: (0, 0))
    return pl.pallas_call(
        body,
        grid=(grid,),
        in_specs=[
            pl.BlockSpec((bt, c), lambda i: (i, 0)),
            pl.BlockSpec((bt, d), lambda i: (i, 0)),
            full((1, d)),
            full((1, d)), full((1, d)), full((d, d)), full((1, d)),
            full((8, d)), full((1, d)), full((d, d)), full((1, d)),
            full((1, d)), full((1, d)), full((1, d)),
        ],
        out_specs=pl.BlockSpec((bt, d), lambda i: (i, 0)),
        out_shape=jax.ShapeDtypeStruct((n, d), jnp.float32),
    )(xr, emb, row0, nw1, nb1, nw2, nb2, tw1, tb1, tw2, tb2, admv, gv, bv)


def kernel(x, table, nw1, nb1, nw2, nb2, tw1, tb1, tw2, tb2, adm, gamma, beta):
    b, s, c = x.shape
    n = b * s
    d = table.shape[1]
    xr = x.reshape(n, c)
    idx2 = _tc_index_prep(xr)
    emb = _sc_embedding_bag(idx2, table, n)
    tw8 = jnp.concatenate([tw1, jnp.zeros((3, d), jnp.float32)], axis=0)
    out = _tc_dense(
        xr, emb, table[0:1],
        nw1, nb1.reshape(1, d), nw2, nb2.reshape(1, d),
        tw8, tb1.reshape(1, d), tw2, tb2.reshape(1, d),
        adm.reshape(1, d), gamma.reshape(1, d), beta.reshape(1, d),
    )
    return out.reshape(b, s, d)


# speedup vs baseline: 1.1576x; 1.0314x over previous
import functools

import jax
import jax.numpy as jnp
from jax import lax
from jax.experimental import pallas as pl
from jax.experimental.pallas import tpu as pltpu
from jax.experimental.pallas import tpu_sc as plsc

_T = 128
_J = 26
_HJ = 13


def _tc_index_prep(xr):
    n, c = xr.shape
    pc = 4
    bt = pc * _T
    grid = n // bt

    def body(x_ref, o_ref):
        ci = x_ref[:, 6:32].astype(jnp.int32)
        cit = ci.T
        for cc in range(pc):
            o_ref[cc * _J:(cc + 1) * _J, :] = cit[:, cc * _T:(cc + 1) * _T]

    return pl.pallas_call(
        body,
        grid=(grid,),
        in_specs=[pl.BlockSpec((bt, c), lambda i: (i, 0))],
        out_specs=pl.BlockSpec((pc * _J, 128), lambda i: (i, 0)),
        out_shape=jax.ShapeDtypeStruct((n // _T * _J, 128), jnp.int32),
    )(xr)


def _sc_embedding_bag(idx2, table, n_tokens):
    info = plsc.get_sparse_core_info()
    nw = info.num_cores * info.num_subcores
    d = table.shape[1]
    g_total = n_tokens // _T
    _HT = _T // 2

    @functools.partial(
        pl.kernel,
        out_type=jax.ShapeDtypeStruct((n_tokens, d), jnp.float32),
        mesh=plsc.VectorSubcoreMesh(core_axis_name="c", subcore_axis_name="s"),
        compiler_params=pltpu.CompilerParams(use_tc_tiling_on_sc=False),
        scratch_types=[
            pltpu.VMEM((_J, 128), jnp.int32),
            pltpu.VMEM((_J, 128), jnp.int32),
            pltpu.VMEM((_HJ, _HT, d), jnp.float32),
            pltpu.VMEM((_HJ, _HT, d), jnp.float32),
            pltpu.VMEM((_T, d), jnp.float32),
            pltpu.SemaphoreType.DMA,
            pltpu.SemaphoreType.DMA,
            pltpu.SemaphoreType.DMA,
        ],
    )
    def k(idx_h, table_h, out_h, idx_a, idx_b, rows_x, rows_y, emb_v,
          sem_x, sem_y, sem_i):
        wid = lax.axis_index("c") * info.num_subcores + lax.axis_index("s")
        base_chunk = jnp.where(wid < 16, wid * 13, 208 + (wid - 16) * 12)

        idx_bufs = (idx_a, idx_b)
        row_bufs = (rows_x, rows_y)
        row_sems = (sem_x, sem_y)

        def issue(i):
            idx_v = idx_bufs[i // 4]
            buf, sem = row_bufs[i % 2], row_sems[i % 2]
            h, p = (i % 4) // 2, i % 2
            return [
                pltpu.async_copy(
                    table_h.at[idx_v.at[p * _HJ + j, pl.ds(h * _HT, _HT)]],
                    buf.at[j],
                    sem,
                )
                for j in range(_HJ)
            ]

        def accumulate(i):
            buf = row_bufs[i % 2]
            h, p = (i % 4) // 2, i % 2

            def tok(t, carry2):
                for q in range(d // 16):
                    if p == 0:
                        a = buf[0, t, pl.ds(q * 16, 16)]
                        jj = range(1, _HJ)
                    else:
                        a = emb_v[h * _HT + t, pl.ds(q * 16, 16)]
                        jj = range(_HJ)
                    for j in jj:
                        a = a + buf[j, t, pl.ds(q * 16, 16)]
                    emb_v[h * _HT + t, pl.ds(q * 16, 16)] = a
                return carry2

            lax.fori_loop(0, _HT, tok, 0)

        def run_chunk_pair(g0, single):
            cpi_a = pltpu.async_copy(
                idx_h.at[pl.ds(g0 * _J, _J)], idx_a, sem_i)
            if not single:
                cpi_b = pltpu.async_copy(
                    idx_h.at[pl.ds((g0 + 1) * _J, _J)], idx_b, sem_i)
            cpi_a.wait()
            nu = 4 if single else 8
            cps = [None] * nu
            cps[0] = issue(0)
            cps[1] = issue(1)
            waited_b = single
            for i in range(nu):
                for cp in cps[i]:
                    cp.wait()
                accumulate(i)
                if i % 4 == 3:
                    g = g0 + i // 4
                    pltpu.sync_copy(emb_v, out_h.at[pl.ds(g * _T, _T)])
                if i + 2 < nu:
                    if not waited_b and i + 2 >= 4:
                        cpi_b.wait()
                        waited_b = True
                    cps[i + 2] = issue(i + 2)

        def pair_body(c2, carry):
            run_chunk_pair(base_chunk + 2 * c2, False)
            return carry

        lax.fori_loop(0, 6, pair_body, 0)

        @pl.when(wid < 16)
        def _():
            run_chunk_pair(base_chunk + 12, True)

    return k(idx2, table)


def _tc_dense(xr, emb, row0, nw1, nb1, nw2, nb2, tw1, tb1, tw2, tb2,
              admv, gv, bv):
    n, c = xr.shape
    d = emb.shape[1]
    bt = 1024
    grid = n // bt

    def body(x_ref, e_ref, r0_r, nw1_r, nb1_r, nw2_r, nb2_r, tw1_r, tb1_r,
             tw2_r, tb2_r, adm_r, g_r, b_r, o_ref):
        xs = x_ref[...]
        ci = xs[:, 6:32]
        cnt = jnp.sum(jnp.where(ci == 0.0, 1.0, 0.0), axis=1, keepdims=True)
        emb_b = e_ref[...] - cnt * r0_r[...]

        num = xs[:, 5:6]
        nmask = jnp.isnan(num)
        numc = jnp.where(nmask, 0.0, num)
        h1 = jnp.maximum(numc * nw1_r[...] + nb1_r[...], 0.0)
        no = jnp.dot(h1, nw2_r[...], preferred_element_type=jnp.float32)
        no = jnp.where(nmask, 0.0, no + nb2_r[...])

        td8 = xs[:, 0:8]
        tmask = jnp.isnan(td8[:, 0:1])
        tdc8 = jnp.where(jnp.isnan(td8), 0.0, td8)
        acc = jnp.dot(tdc8, tw1_r[...],
                      preferred_element_type=jnp.float32) + tb1_r[...]
        h2 = jnp.maximum(acc, 0.0)
        to = jnp.dot(h2, tw2_r[...], preferred_element_type=jnp.float32)
        to = jnp.where(tmask, 0.0, to + tb2_r[...])

        enc = emb_b + no + to
        nrm = jnp.sqrt(jnp.sum(enc * enc, axis=1, keepdims=True))
        enc = enc / jnp.maximum(nrm, 1e-10)
        enc = enc + xs[:, 32:33] * adm_r[...]
        mu = jnp.mean(enc, axis=1, keepdims=True)
        dev = enc - mu
        var = jnp.mean(dev * dev, axis=1, keepdims=True)
        o_ref[...] = dev * lax.rsqrt(var + 1e-5) * g_r[...] + b_r[...]

    full = lambda shape: pl.BlockSpec(shape, lambda i: (0, 0))
    return pl.pallas_call(
        body,
        grid=(grid,),
        in_specs=[
            pl.BlockSpec((bt, c), lambda i: (i, 0)),
            pl.BlockSpec((bt, d), lambda i: (i, 0)),
            full((1, d)),
            full((1, d)), full((1, d)), full((d, d)), full((1, d)),
            full((8, d)), full((1, d)), full((d, d)), full((1, d)),
            full((1, d)), full((1, d)), full((1, d)),
        ],
        out_specs=pl.BlockSpec((bt, d), lambda i: (i, 0)),
        out_shape=jax.ShapeDtypeStruct((n, d), jnp.float32),
    )(xr, emb, row0, nw1, nb1, nw2, nb2, tw1, tb1, tw2, tb2, admv, gv, bv)


def kernel(x, table, nw1, nb1, nw2, nb2, tw1, tb1, tw2, tb2, adm, gamma, beta):
    b, s, c = x.shape
    n = b * s
    d = table.shape[1]
    xr = x.reshape(n, c)
    idx2 = _tc_index_prep(xr)
    emb = _sc_embedding_bag(idx2, table, n)
    tw8 = jnp.concatenate([tw1, jnp.zeros((3, d), jnp.float32)], axis=0)
    out = _tc_dense(
        xr, emb, table[0:1],
        nw1, nb1.reshape(1, d), nw2, nb2.reshape(1, d),
        tw8, tb1.reshape(1, d), tw2, tb2.reshape(1, d),
        adm.reshape(1, d), gamma.reshape(1, d), beta.reshape(1, d),
    )
    return out.reshape(b, s, d)


# speedup vs baseline: 1.1756x; 1.0156x over previous
import functools

import jax
import jax.numpy as jnp
from jax import lax
from jax.experimental import pallas as pl
from jax.experimental.pallas import tpu as pltpu
from jax.experimental.pallas import tpu_sc as plsc

_T = 128
_J = 26
_HJ = 13


def _tc_index_prep(xr):
    n, c = xr.shape
    pc = 4
    bt = pc * _T
    grid = n // bt

    def body(x_ref, o_ref):
        ci = x_ref[:, 6:32].astype(jnp.int32)
        cit = ci.T
        for cc in range(pc):
            o_ref[cc * _J:(cc + 1) * _J, :] = cit[:, cc * _T:(cc + 1) * _T]

    return pl.pallas_call(
        body,
        grid=(grid,),
        in_specs=[pl.BlockSpec((bt, c), lambda i: (i, 0))],
        out_specs=pl.BlockSpec((pc * _J, 128), lambda i: (i, 0)),
        out_shape=jax.ShapeDtypeStruct((n // _T * _J, 128), jnp.int32),
    )(xr)


def _sc_embedding_bag(idx2, table, n_tokens):
    info = plsc.get_sparse_core_info()
    nw = info.num_cores * info.num_subcores
    d = table.shape[1]
    g_total = n_tokens // _T
    _HT = _T // 2

    @functools.partial(
        pl.kernel,
        out_type=jax.ShapeDtypeStruct((n_tokens, d), jnp.float32),
        mesh=plsc.VectorSubcoreMesh(core_axis_name="c", subcore_axis_name="s"),
        compiler_params=pltpu.CompilerParams(use_tc_tiling_on_sc=False),
        scratch_types=[
            pltpu.VMEM((_J, 128), jnp.int32),
            pltpu.VMEM((_J, 128), jnp.int32),
            pltpu.VMEM((_HJ, _HT, d), jnp.float32),
            pltpu.VMEM((_HJ, _HT, d), jnp.float32),
            pltpu.VMEM((_T, d), jnp.float32),
            pltpu.SemaphoreType.DMA,
            pltpu.SemaphoreType.DMA,
            pltpu.SemaphoreType.DMA,
        ],
    )
    def k(idx_h, table_h, out_h, idx_a, idx_b, rows_x, rows_y, emb_v,
          sem_x, sem_y, sem_i):
        wid = lax.axis_index("c") * info.num_subcores + lax.axis_index("s")
        base_chunk = jnp.where(wid < 16, wid * 13, 208 + (wid - 16) * 12)

        idx_bufs = (idx_a, idx_b)
        row_bufs = (rows_x, rows_y)
        row_sems = (sem_x, sem_y)

        def issue(i):
            idx_v = idx_bufs[i // 4]
            buf, sem = row_bufs[i % 2], row_sems[i % 2]
            h, p = (i % 4) // 2, i % 2
            return [
                pltpu.async_copy(
                    table_h.at[idx_v.at[p * _HJ + j, pl.ds(h * _HT, _HT)]],
                    buf.at[j],
                    sem,
                )
                for j in range(_HJ)
            ]

        def accumulate(i):
            buf = row_bufs[i % 2]
            h, p = (i % 4) // 2, i % 2

            def tok(t, carry2):
                for q in range(d // 16):
                    if p == 0:
                        a = buf[0, t, pl.ds(q * 16, 16)]
                        jj = range(1, _HJ)
                    else:
                        a = emb_v[h * _HT + t, pl.ds(q * 16, 16)]
                        jj = range(_HJ)
                    for j in jj:
                        a = a + buf[j, t, pl.ds(q * 16, 16)]
                    emb_v[h * _HT + t, pl.ds(q * 16, 16)] = a
                return carry2

            lax.fori_loop(0, _HT, tok, 0)

        def run_chunk_pair(g0, single):
            cpi_a = pltpu.async_copy(
                idx_h.at[pl.ds(g0 * _J, _J)], idx_a, sem_i)
            if not single:
                cpi_b = pltpu.async_copy(
                    idx_h.at[pl.ds((g0 + 1) * _J, _J)], idx_b, sem_i)
            cpi_a.wait()
            nu = 4 if single else 8
            cps = [None] * nu
            cps[0] = issue(0)
            cps[1] = issue(1)
            waited_b = single
            for i in range(nu):
                for cp in cps[i]:
                    cp.wait()
                accumulate(i)
                if i % 4 == 3:
                    g = g0 + i // 4
                    pltpu.sync_copy(emb_v, out_h.at[pl.ds(g * _T, _T)])
                if i + 2 < nu:
                    if not waited_b and i + 2 >= 4:
                        cpi_b.wait()
                        waited_b = True
                    cps[i + 2] = issue(i + 2)

        def pair_body(c2, carry):
            run_chunk_pair(base_chunk + 2 * c2, False)
            return carry

        lax.fori_loop(0, 6, pair_body, 0)

        @pl.when(wid < 16)
        def _():
            run_chunk_pair(base_chunk + 12, True)

    return k(idx2, table)


def _tc_dense(xr, emb, r26, nw1, nb1, nw2, nb2, tw1, tb1, tw2, tb2,
              admv, gv, bv):
    n, c = xr.shape
    d = emb.shape[1]
    bt = 2048
    grid = n // bt
    o64c = jnp.ones((d, 1), jnp.float32)
    o64r = jnp.ones((1, d), jnp.float32)

    def mm(a, b):
        return jnp.dot(a, b, preferred_element_type=jnp.float32)

    def body(x_ref, e_ref, oc_r, or_r, r26_r, nw1_r, nb1_r, nw2_r, nb2_r,
             tw1_r, tb1_r, tw2_r, tb2_r, adm_r, g_r, b_r, o_ref):
        xs = x_ref[...]
        mask26 = jnp.where(xs[:, 6:32] == 0.0, 1.0, 0.0)
        emb_b = e_ref[...] - mm(mask26, r26_r[...])

        num = xs[:, 5:6]
        nvalid = jnp.where(jnp.isnan(num), 0.0, 1.0)
        numc = jnp.where(jnp.isnan(num), 0.0, num)
        h1 = jnp.maximum(mm(numc, nw1_r[...]) + nb1_r[...], 0.0)
        no = (mm(h1, nw2_r[...]) + nb2_r[...]) * mm(nvalid, or_r[...])

        td8 = xs[:, 0:8]
        tvalid = jnp.where(jnp.isnan(td8[:, 0:1]), 0.0, 1.0)
        tdc8 = jnp.where(jnp.isnan(td8), 0.0, td8)
        h2 = jnp.maximum(mm(tdc8, tw1_r[...]) + tb1_r[...], 0.0)
        to = (mm(h2, tw2_r[...]) + tb2_r[...]) * mm(tvalid, or_r[...])

        enc = emb_b + no + to
        m1 = mm(enc * enc, oc_r[...])
        inv = 1.0 / jnp.maximum(jnp.sqrt(m1), 1e-10)
        enc = enc * mm(inv, or_r[...])
        enc = enc + mm(xs[:, 32:33], adm_r[...])
        mu = mm(enc, oc_r[...]) * (1.0 / d)
        dev = enc - mm(mu, or_r[...])
        var = mm(dev * dev, oc_r[...]) * (1.0 / d)
        iv = lax.rsqrt(var + 1e-5)
        o_ref[...] = dev * mm(iv, or_r[...]) * g_r[...] + b_r[...]

    full = lambda shape: pl.BlockSpec(shape, lambda i: (0, 0))
    return pl.pallas_call(
        body,
        grid=(grid,),
        in_specs=[
            pl.BlockSpec((bt, c), lambda i: (i, 0)),
            pl.BlockSpec((bt, d), lambda i: (i, 0)),
            full((d, 1)), full((1, d)), full((26, d)),
            full((1, d)), full((1, d)), full((d, d)), full((1, d)),
            full((8, d)), full((1, d)), full((d, d)), full((1, d)),
            full((1, d)), full((1, d)), full((1, d)),
        ],
        out_specs=pl.BlockSpec((bt, d), lambda i: (i, 0)),
        out_shape=jax.ShapeDtypeStruct((n, d), jnp.float32),
    )(xr, emb, o64c, o64r, r26, nw1, nb1, nw2, nb2, tw1, tb1, tw2, tb2,
      admv, gv, bv)


def kernel(x, table, nw1, nb1, nw2, nb2, tw1, tb1, tw2, tb2, adm, gamma, beta):
    b, s, c = x.shape
    n = b * s
    d = table.shape[1]
    xr = x.reshape(n, c)
    idx2 = _tc_index_prep(xr)
    emb = _sc_embedding_bag(idx2, table, n)
    tw8 = jnp.concatenate([tw1, jnp.zeros((3, d), jnp.float32)], axis=0)
    out = _tc_dense(
        xr, emb, jnp.tile(table[0:1], (26, 1)),
        nw1, nb1.reshape(1, d), nw2, nb2.reshape(1, d),
        tw8, tb1.reshape(1, d), tw2, tb2.reshape(1, d),
        adm.reshape(1, d), gamma.reshape(1, d), beta.reshape(1, d),
    )
    return out.reshape(b, s, d)
